# Initial kernel scaffold; baseline (speedup 1.0000x reference)
import jax
import jax.numpy as jnp
from jax.experimental import pallas as pl


def kernel(input_ids, label, embed_table, W, b):
    raise NotImplementedError("write your pallas kernel here")


# speedup vs baseline: 5.5528x; 5.5528x over previous
import functools

import jax
import jax.numpy as jnp
from jax import lax
from jax.experimental import pallas as pl
from jax.experimental.pallas import tpu as pltpu
from jax.experimental.pallas import tpu_sc as plsc

_L = 16


def _splat(val_ref, flat_idx):
    return plsc.load_gather(
        val_ref, [jnp.full((_L,), flat_idx, dtype=jnp.int32)]
    )


def _make_sc_call(n_total, n_workers, chunk, v_rows, d_in, d_out):
    n_per_worker = n_total // n_workers
    n_chunks = n_per_worker // chunk
    groups = chunk // _L

    mesh = plsc.VectorSubcoreMesh(
        core_axis_name="c", subcore_axis_name="s", num_cores=2, num_subcores=16
    )
    tbl_words = max(((v_rows * d_out + 127) // 128) * 128, 128)

    @functools.partial(
        pl.kernel,
        out_type=jax.ShapeDtypeStruct((n_total * d_out,), jnp.float32),
        mesh=mesh,
        compiler_params=pltpu.CompilerParams(needs_layout_passes=False),
        scratch_types=[
            pltpu.VMEM((chunk,), jnp.int32),
            pltpu.VMEM((chunk * d_out,), jnp.float32),
            pltpu.VMEM((tbl_words,), jnp.float32),
            pltpu.VMEM((256,), jnp.float32),
            pltpu.VMEM((128,), jnp.float32),
            pltpu.VMEM((128,), jnp.float32),
        ],
    )
    def sc_call(ids_hbm, et_hbm, w_hbm, b_hbm, out_hbm,
                ids_v, out_v, tbl_v, et_v, w_v, b_v):
        wid = lax.axis_index("s") * 2 + lax.axis_index("c")
        lanes = lax.iota(jnp.int32, _L)

        pltpu.sync_copy(et_hbm, et_v)
        pltpu.sync_copy(w_hbm, w_v)
        pltpu.sync_copy(b_hbm, b_v)

        n_row_groups = (v_rows + _L - 1) // _L
        for grp in range(n_row_groups):
            rows = jnp.minimum(lanes + grp * _L, v_rows - 1)
            accs = [_splat(b_v, 16 + c) for c in range(d_out)]
            for d in range(d_in):
                ecol = plsc.load_gather(et_v, [rows * d_in + d])
                ecol = jnp.maximum(ecol, 0.0)
                for c in range(d_out):
                    accs[c] = accs[c] + ecol * _splat(w_v, 16 + d * d_out + c)
            for c in range(d_out):
                plsc.store_scatter(tbl_v, [rows * d_out + c], accs[c])

        base = wid * n_per_worker
        iota3 = lanes * d_out

        def chunk_body(k, carry):
            off = base + k * chunk
            pltpu.sync_copy(ids_hbm.at[pl.ds(off, chunk)], ids_v)

            def grp_body(g, c2):
                idv = ids_v[pl.ds(g * _L, _L)]
                idx0 = idv * d_out
                ob = g * (_L * d_out)
                for c in range(d_out):
                    vals = plsc.load_gather(tbl_v, [idx0 + c])
                    plsc.store_scatter(out_v, [iota3 + (ob + c)], vals)
                return c2

            lax.fori_loop(0, groups, grp_body, 0)
            pltpu.sync_copy(out_v, out_hbm.at[pl.ds(off * d_out, chunk * d_out)])
            return carry

        lax.fori_loop(0, n_chunks, chunk_body, 0)

    return sc_call


@jax.jit
def kernel(input_ids, label, embed_table, W, b):
    del label
    batch, seq = input_ids.shape
    v_rows, d_in = embed_table.shape
    d_out = W.shape[1]
    n_total = batch * seq

    n_workers = 32
    chunk = 4096
    assert n_total % (n_workers * chunk) == 0

    ids_flat = input_ids.reshape(n_total).astype(jnp.int32)
    et_flat = jnp.zeros((256,), embed_table.dtype).at[: v_rows * d_in].set(
        embed_table.reshape(-1)
    )
    w_flat = jnp.zeros((128,), W.dtype).at[16 : 16 + d_in * d_out].set(W.reshape(-1))
    b_pad = jnp.zeros((128,), b.dtype).at[16 : 16 + d_out].set(b)
    sc_call = _make_sc_call(n_total, n_workers, chunk, v_rows, d_in, d_out)
    out_flat = sc_call(ids_flat, et_flat, w_flat, b_pad)
    return out_flat.reshape(batch, seq, d_out)


# speedup vs baseline: 65.8648x; 11.8616x over previous
import functools

import jax
import jax.numpy as jnp
from jax import lax
from jax.experimental import pallas as pl
from jax.experimental.pallas import tpu as pltpu
from jax.experimental.pallas import tpu_sc as plsc

_L = 16


def _splat(val_ref, flat_idx):
    return plsc.load_gather(
        val_ref, [jnp.full((_L,), flat_idx, dtype=jnp.int32)]
    )


def _make_sc_call(n_total, n_workers, chunk, v_rows, d_in, d_out):
    n_per_worker = n_total // n_workers
    n_chunks = n_per_worker // chunk
    groups = chunk // _L

    mesh = plsc.VectorSubcoreMesh(
        core_axis_name="c", subcore_axis_name="s", num_cores=2, num_subcores=16
    )
    tbl_words = max(((v_rows * d_out + 127) // 128) * 128, 128)

    @functools.partial(
        pl.kernel,
        out_type=jax.ShapeDtypeStruct((d_out * n_total,), jnp.float32),
        mesh=mesh,
        compiler_params=pltpu.CompilerParams(needs_layout_passes=False),
        scratch_types=[
            pltpu.VMEM((chunk,), jnp.int32),
            [pltpu.VMEM((chunk,), jnp.float32) for _ in range(d_out)],
            pltpu.VMEM((tbl_words,), jnp.float32),
            pltpu.VMEM((256,), jnp.float32),
            pltpu.VMEM((128,), jnp.float32),
            pltpu.VMEM((128,), jnp.float32),
        ],
    )
    def sc_call(ids_hbm, et_hbm, w_hbm, b_hbm, out_hbm,
                ids_v, out_vs, tbl_v, et_v, w_v, b_v):
        wid = lax.axis_index("s") * 2 + lax.axis_index("c")
        lanes = lax.iota(jnp.int32, _L)

        pltpu.sync_copy(et_hbm, et_v)
        pltpu.sync_copy(w_hbm, w_v)
        pltpu.sync_copy(b_hbm, b_v)

        n_row_groups = (v_rows + _L - 1) // _L
        for grp in range(n_row_groups):
            rows = jnp.minimum(lanes + grp * _L, v_rows - 1)
            accs = [_splat(b_v, 16 + c) for c in range(d_out)]
            for d in range(d_in):
                ecol = plsc.load_gather(et_v, [rows * d_in + d])
                ecol = jnp.maximum(ecol, 0.0)
                for c in range(d_out):
                    accs[c] = accs[c] + ecol * _splat(w_v, 16 + d * d_out + c)
            for c in range(d_out):
                plsc.store_scatter(tbl_v, [rows * d_out + c], accs[c])

        base = wid * n_per_worker

        def chunk_body(k, carry):
            off = base + k * chunk
            pltpu.sync_copy(ids_hbm.at[pl.ds(off, chunk)], ids_v)

            def grp_body(g, c2):
                sl = pl.ds(g * _L, _L)
                idx0 = ids_v[sl] * d_out
                for c in range(d_out):
                    out_vs[c][sl] = plsc.load_gather(tbl_v, [idx0 + c])
                return c2

            lax.fori_loop(0, groups, grp_body, 0)
            for c in range(d_out):
                pltpu.sync_copy(
                    out_vs[c], out_hbm.at[pl.ds(c * n_total + off, chunk)]
                )
            return carry

        lax.fori_loop(0, n_chunks, chunk_body, 0)

    return sc_call


@jax.jit
def kernel(input_ids, label, embed_table, W, b):
    del label
    batch, seq = input_ids.shape
    v_rows, d_in = embed_table.shape
    d_out = W.shape[1]
    n_total = batch * seq

    n_workers = 32
    chunk = 4096
    assert n_total % (n_workers * chunk) == 0

    ids_flat = input_ids.T.reshape(n_total).astype(jnp.int32)
    et_flat = jnp.zeros((256,), embed_table.dtype).at[: v_rows * d_in].set(
        embed_table.reshape(-1)
    )
    w_flat = jnp.zeros((128,), W.dtype).at[16 : 16 + d_in * d_out].set(W.reshape(-1))
    b_pad = jnp.zeros((128,), b.dtype).at[16 : 16 + d_out].set(b)
    sc_call = _make_sc_call(n_total, n_workers, chunk, v_rows, d_in, d_out)
    out_flat = sc_call(ids_flat, et_flat, w_flat, b_pad)
    return out_flat.reshape(d_out, seq, batch).transpose(2, 1, 0)


# speedup vs baseline: 118.5053x; 1.7992x over previous
import functools

import jax
import jax.numpy as jnp
from jax import lax
from jax.experimental import pallas as pl
from jax.experimental.pallas import tpu as pltpu
from jax.experimental.pallas import tpu_sc as plsc

_L = 16


def _splat(val_ref, flat_idx):
    return plsc.load_gather(
        val_ref, [jnp.full((_L,), flat_idx, dtype=jnp.int32)]
    )


def _make_sc_call(n_total, n_workers, chunk, v_rows, d_in, d_out):
    n_per_worker = n_total // n_workers
    n_chunks = n_per_worker // chunk
    groups = chunk // _L

    mesh = plsc.VectorSubcoreMesh(
        core_axis_name="c", subcore_axis_name="s", num_cores=2, num_subcores=16
    )
    tbl_words = max(((v_rows * d_out + 127) // 128) * 128, 128)

    @functools.partial(
        pl.kernel,
        out_type=jax.ShapeDtypeStruct((d_out * n_total,), jnp.float32),
        mesh=mesh,
        compiler_params=pltpu.CompilerParams(needs_layout_passes=False),
        scratch_types=[
            [pltpu.VMEM((chunk,), jnp.int32) for _ in range(2)],
            [[pltpu.VMEM((chunk,), jnp.float32) for _ in range(d_out)]
             for _ in range(2)],
            pltpu.VMEM((tbl_words,), jnp.float32),
            pltpu.VMEM((256,), jnp.float32),
            pltpu.VMEM((128,), jnp.float32),
            pltpu.VMEM((128,), jnp.float32),
            [pltpu.SemaphoreType.DMA for _ in range(2)],
            [pltpu.SemaphoreType.DMA for _ in range(2)],
            pltpu.SemaphoreType.DMA,
        ],
    )
    def sc_call(ids_hbm, et_hbm, w_hbm, b_hbm, out_hbm,
                ids_v, out_vs, tbl_v, et_v, w_v, b_v,
                sem_in, sem_out, sem_p):
        wid = lax.axis_index("s") * 2 + lax.axis_index("c")
        lanes = lax.iota(jnp.int32, _L)
        base = wid * n_per_worker

        in_descs = [None, None]
        in_descs[0] = pltpu.async_copy(
            ids_hbm.at[pl.ds(base, chunk)], ids_v[0], sem_in[0]
        )

        p0 = pltpu.async_copy(et_hbm, et_v, sem_p)
        p1 = pltpu.async_copy(w_hbm, w_v, sem_p)
        p2 = pltpu.async_copy(b_hbm, b_v, sem_p)
        p0.wait()
        p1.wait()
        p2.wait()

        n_row_groups = (v_rows + _L - 1) // _L
        for grp in range(n_row_groups):
            rows = jnp.minimum(lanes + grp * _L, v_rows - 1)
            accs = [_splat(b_v, 16 + c) for c in range(d_out)]
            for d in range(d_in):
                ecol = plsc.load_gather(et_v, [rows * d_in + d])
                ecol = jnp.maximum(ecol, 0.0)
                for c in range(d_out):
                    accs[c] = accs[c] + ecol * _splat(w_v, 16 + d * d_out + c)
            for c in range(d_out):
                plsc.store_scatter(tbl_v, [rows * d_out + c], accs[c])

        out_descs = [None, None]
        for k in range(n_chunks):
            buf = k % 2
            off = base + k * chunk
            if k + 1 < n_chunks:
                in_descs[1 - buf] = pltpu.async_copy(
                    ids_hbm.at[pl.ds(off + chunk, chunk)],
                    ids_v[1 - buf],
                    sem_in[1 - buf],
                )
            in_descs[buf].wait()
            if out_descs[buf] is not None:
                for dsc in out_descs[buf]:
                    dsc.wait()

            ids_b = ids_v[buf]
            out_b = out_vs[buf]

            @plsc.parallel_loop(0, chunk, step=_L, unroll=8)
            def grp_body(i):
                sl = pl.ds(i, _L)
                idx0 = ids_b[sl] * d_out
                for c in range(d_out):
                    out_b[c][sl] = plsc.load_gather(tbl_v, [idx0 + c])

            out_descs[buf] = [
                pltpu.async_copy(
                    out_b[c],
                    out_hbm.at[pl.ds(c * n_total + off, chunk)],
                    sem_out[buf],
                )
                for c in range(d_out)
            ]
        for buf in range(2):
            if out_descs[buf] is not None:
                for dsc in out_descs[buf]:
                    dsc.wait()

    return sc_call


@jax.jit
def kernel(input_ids, label, embed_table, W, b):
    del label
    batch, seq = input_ids.shape
    v_rows, d_in = embed_table.shape
    d_out = W.shape[1]
    n_total = batch * seq

    n_workers = 32
    chunk = 4096
    assert n_total % (n_workers * chunk) == 0

    ids_flat = input_ids.T.reshape(n_total).astype(jnp.int32)
    et_flat = jnp.zeros((256,), embed_table.dtype).at[: v_rows * d_in].set(
        embed_table.reshape(-1)
    )
    w_flat = jnp.zeros((128,), W.dtype).at[16 : 16 + d_in * d_out].set(W.reshape(-1))
    b_pad = jnp.zeros((128,), b.dtype).at[16 : 16 + d_out].set(b)
    sc_call = _make_sc_call(n_total, n_workers, chunk, v_rows, d_in, d_out)
    out_flat = sc_call(ids_flat, et_flat, w_flat, b_pad)
    return out_flat.reshape(d_out, seq, batch).transpose(2, 1, 0)


# speedup vs baseline: 257.9965x; 2.1771x over previous
import functools

import jax
import jax.numpy as jnp
from jax import lax
from jax.experimental import pallas as pl
from jax.experimental.pallas import tpu as pltpu
from jax.experimental.pallas import tpu_sc as plsc

_L = 16


def _splat(val_ref, flat_idx):
    return plsc.load_gather(
        val_ref, [jnp.full((_L,), flat_idx, dtype=jnp.int32)]
    )


def _make_sc_call(n_total, n_workers, chunk, v_rows, d_in, d_out):
    n_per_worker = n_total // n_workers
    n_chunks = n_per_worker // chunk
    groups = chunk // _L

    mesh = plsc.VectorSubcoreMesh(
        core_axis_name="c", subcore_axis_name="s", num_cores=2, num_subcores=16
    )
    tbl_words = max(((v_rows * d_out + 127) // 128) * 128, 128)

    @functools.partial(
        pl.kernel,
        out_type=jax.ShapeDtypeStruct((d_out * n_total,), jnp.float32),
        mesh=mesh,
        compiler_params=pltpu.CompilerParams(needs_layout_passes=False),
        scratch_types=[
            [pltpu.VMEM((chunk,), jnp.int32) for _ in range(2)],
            [[pltpu.VMEM((chunk,), jnp.float32) for _ in range(d_out)]
             for _ in range(2)],
            pltpu.VMEM((tbl_words,), jnp.float32),
            pltpu.VMEM((256,), jnp.float32),
            pltpu.VMEM((128,), jnp.float32),
            pltpu.VMEM((128,), jnp.float32),
            [pltpu.SemaphoreType.DMA for _ in range(2)],
            [pltpu.SemaphoreType.DMA for _ in range(2)],
            pltpu.SemaphoreType.DMA,
        ],
    )
    def sc_call(ids_hbm, et_hbm, w_hbm, b_hbm, out_hbm,
                ids_v, out_vs, tbl_v, et_v, w_v, b_v,
                sem_in, sem_out, sem_p):
        wid = lax.axis_index("s") * 2 + lax.axis_index("c")
        lanes = lax.iota(jnp.int32, _L)
        base = wid * n_per_worker

        in_descs = [None, None]
        in_descs[0] = pltpu.async_copy(
            ids_hbm.at[pl.ds(base, chunk)], ids_v[0], sem_in[0]
        )

        p0 = pltpu.async_copy(et_hbm, et_v, sem_p)
        p1 = pltpu.async_copy(w_hbm, w_v, sem_p)
        p2 = pltpu.async_copy(b_hbm, b_v, sem_p)
        p0.wait()
        p1.wait()
        p2.wait()

        n_row_groups = (v_rows + _L - 1) // _L
        for grp in range(n_row_groups):
            rows = jnp.minimum(lanes + grp * _L, v_rows - 1)
            accs = [_splat(b_v, 16 + c) for c in range(d_out)]
            for d in range(d_in):
                ecol = plsc.load_gather(et_v, [rows * d_in + d])
                ecol = jnp.maximum(ecol, 0.0)
                for c in range(d_out):
                    accs[c] = accs[c] + ecol * _splat(w_v, 16 + d * d_out + c)
            for c in range(d_out):
                plsc.store_scatter(tbl_v, [rows * d_out + c], accs[c])

        out_descs = [None, None]
        for k in range(n_chunks):
            buf = k % 2
            off = base + k * chunk
            if k + 1 < n_chunks:
                in_descs[1 - buf] = pltpu.async_copy(
                    ids_hbm.at[pl.ds(off + chunk, chunk)],
                    ids_v[1 - buf],
                    sem_in[1 - buf],
                )
            in_descs[buf].wait()
            if out_descs[buf] is not None:
                for dsc in out_descs[buf]:
                    dsc.wait()

            ids_b = ids_v[buf]
            out_b = out_vs[buf]

            @plsc.parallel_loop(0, chunk, step=_L, unroll=8)
            def grp_body(i):
                sl = pl.ds(i, _L)
                idx0 = ids_b[sl] * d_out
                for c in range(d_out):
                    out_b[c][sl] = plsc.load_gather(tbl_v, [idx0 + c])

            out_descs[buf] = [
                pltpu.async_copy(
                    out_b[c],
                    out_hbm.at[pl.ds(c * n_total + off, chunk)],
                    sem_out[buf],
                )
                for c in range(d_out)
            ]
        for buf in range(2):
            if out_descs[buf] is not None:
                for dsc in out_descs[buf]:
                    dsc.wait()

    return sc_call


@jax.jit
def kernel(input_ids, label, embed_table, W, b):
    del label
    batch, seq = input_ids.shape
    v_rows, d_in = embed_table.shape
    d_out = W.shape[1]
    n_total = batch * seq

    n_workers = 32
    chunk = 4096
    assert n_total % (n_workers * chunk) == 0

    jt, js = seq // 8, 8
    it, il = batch // 128, 128
    ids_flat = (
        input_ids.T.reshape(jt, js, it, il)
        .transpose(0, 2, 1, 3)
        .reshape(n_total)
        .astype(jnp.int32)
    )
    et_flat = jnp.zeros((256,), embed_table.dtype).at[: v_rows * d_in].set(
        embed_table.reshape(-1)
    )
    w_flat = jnp.zeros((128,), W.dtype).at[16 : 16 + d_in * d_out].set(W.reshape(-1))
    b_pad = jnp.zeros((128,), b.dtype).at[16 : 16 + d_out].set(b)
    sc_call = _make_sc_call(n_total, n_workers, chunk, v_rows, d_in, d_out)
    out_flat = sc_call(ids_flat, et_flat, w_flat, b_pad)
    return (
        out_flat.reshape(d_out, jt, it, js, il)
        .transpose(2, 4, 1, 3, 0)
        .reshape(batch, seq, d_out)
    )


# speedup vs baseline: 295.7769x; 1.1464x over previous
import functools

import jax
import jax.numpy as jnp
from jax import lax
from jax.experimental import pallas as pl
from jax.experimental.pallas import tpu as pltpu
from jax.experimental.pallas import tpu_sc as plsc

_L = 16


def _splat(val_ref, flat_idx):
    return plsc.load_gather(
        val_ref, [jnp.full((_L,), flat_idx, dtype=jnp.int32)]
    )


def _make_sc_call(n_total, n_workers, chunk, v_rows, d_in, d_out):
    n_per_worker = n_total // n_workers
    n_chunks = n_per_worker // chunk
    groups = chunk // _L

    mesh = plsc.VectorSubcoreMesh(
        core_axis_name="c", subcore_axis_name="s", num_cores=2, num_subcores=16
    )
    tbl_words = max(((v_rows * d_out + 127) // 128) * 128, 128)

    @functools.partial(
        pl.kernel,
        out_type=jax.ShapeDtypeStruct((d_out * n_total,), jnp.float32),
        mesh=mesh,
        compiler_params=pltpu.CompilerParams(needs_layout_passes=False),
        scratch_types=[
            [pltpu.VMEM((chunk,), jnp.int32) for _ in range(2)],
            [[pltpu.VMEM((chunk,), jnp.float32) for _ in range(d_out)]
             for _ in range(2)],
            pltpu.VMEM((tbl_words,), jnp.float32),
            pltpu.VMEM((256,), jnp.float32),
            pltpu.VMEM((128,), jnp.float32),
            pltpu.VMEM((128,), jnp.float32),
            [pltpu.SemaphoreType.DMA for _ in range(2)],
            [pltpu.SemaphoreType.DMA for _ in range(2)],
            pltpu.SemaphoreType.DMA,
        ],
    )
    def sc_call(ids_hbm, et_hbm, w_hbm, b_hbm, out_hbm,
                ids_v, out_vs, tbl_v, et_v, w_v, b_v,
                sem_in, sem_out, sem_p):
        wid = lax.axis_index("s") * 2 + lax.axis_index("c")
        lanes = lax.iota(jnp.int32, _L)
        base = wid * n_per_worker

        in_descs = [None, None]
        in_descs[0] = pltpu.async_copy(
            ids_hbm.at[pl.ds(base, chunk)], ids_v[0], sem_in[0]
        )

        p0 = pltpu.async_copy(et_hbm, et_v, sem_p)
        p1 = pltpu.async_copy(w_hbm, w_v, sem_p)
        p2 = pltpu.async_copy(b_hbm, b_v, sem_p)
        p0.wait()
        p1.wait()
        p2.wait()

        n_row_groups = (v_rows + _L - 1) // _L
        for grp in range(n_row_groups):
            rows = jnp.minimum(lanes + grp * _L, v_rows - 1)
            accs = [_splat(b_v, 16 + c) for c in range(d_out)]
            for d in range(d_in):
                ecol = plsc.load_gather(et_v, [rows * d_in + d])
                ecol = jnp.maximum(ecol, 0.0)
                for c in range(d_out):
                    accs[c] = accs[c] + ecol * _splat(w_v, 16 + d * d_out + c)
            for c in range(d_out):
                plsc.store_scatter(tbl_v, [rows * d_out + c], accs[c])

        out_descs = [None, None]
        for k in range(n_chunks):
            buf = k % 2
            off = base + k * chunk
            if k + 1 < n_chunks:
                in_descs[1 - buf] = pltpu.async_copy(
                    ids_hbm.at[pl.ds(off + chunk, chunk)],
                    ids_v[1 - buf],
                    sem_in[1 - buf],
                )
            in_descs[buf].wait()
            if out_descs[buf] is not None:
                for dsc in out_descs[buf]:
                    dsc.wait()

            ids_b = ids_v[buf]
            out_b = out_vs[buf]

            @plsc.parallel_loop(0, chunk, step=_L, unroll=8)
            def grp_body(i):
                sl = pl.ds(i, _L)
                idx0 = ids_b[sl] * d_out
                for c in range(d_out):
                    out_b[c][sl] = plsc.load_gather(tbl_v, [idx0 + c])

            out_descs[buf] = [
                pltpu.async_copy(
                    out_b[c],
                    out_hbm.at[pl.ds(c * n_total + off, chunk)],
                    sem_out[buf],
                )
                for c in range(d_out)
            ]
        for buf in range(2):
            if out_descs[buf] is not None:
                for dsc in out_descs[buf]:
                    dsc.wait()

    return sc_call


@jax.jit
def kernel(input_ids, label, embed_table, W, b):
    del label
    batch, seq = input_ids.shape
    v_rows, d_in = embed_table.shape
    d_out = W.shape[1]
    n_total = batch * seq

    n_workers = 32
    chunk = 10240
    assert n_total % (n_workers * chunk) == 0

    jt, js = seq // 8, 8
    it, il = batch // 128, 128
    ids_flat = (
        input_ids.T.reshape(jt, js, it, il)
        .transpose(0, 2, 1, 3)
        .reshape(n_total)
        .astype(jnp.int32)
    )
    et_flat = jnp.zeros((256,), embed_table.dtype).at[: v_rows * d_in].set(
        embed_table.reshape(-1)
    )
    w_flat = jnp.zeros((128,), W.dtype).at[16 : 16 + d_in * d_out].set(W.reshape(-1))
    b_pad = jnp.zeros((128,), b.dtype).at[16 : 16 + d_out].set(b)
    sc_call = _make_sc_call(n_total, n_workers, chunk, v_rows, d_in, d_out)
    out_flat = sc_call(ids_flat, et_flat, w_flat, b_pad)
    return (
        out_flat.reshape(d_out, jt, it, js, il)
        .transpose(2, 4, 1, 3, 0)
        .reshape(batch, seq, d_out)
    )
